# baseline (speedup 1.0000x reference)
import functools

import jax
import jax.numpy as jnp
from jax import lax
from jax.experimental import pallas as pl
from jax.experimental.pallas import tpu as pltpu
from jax.experimental.pallas import tpu_sc as plsc

B = 16384
D = 64
K = 20
V = 1000000
NC = 2
NS = 16
NW = NC * NS
CHUNK = 128

VFULL = (V // CHUNK) * CHUNK
NBLK = V // CHUNK
VPAIR = V // 2

CA = B // NW // CHUNK
CP = B // NW // CHUNK
CN = B * K // NW // CHUNK


def _sc_detile():
    mesh = plsc.VectorSubcoreMesh(core_axis_name="c", subcore_axis_name="s")
    CW = 384
    OW = CW // 2
    NB = V // CW
    TOT = 2 * NB
    TMAX = 2 * ((TOT // NW) // 2 + 2)

    @functools.partial(
        pl.kernel,
        out_type=(
            jax.ShapeDtypeStruct((VPAIR, CHUNK), jnp.float32),
            jax.ShapeDtypeStruct((VPAIR, CHUNK), jnp.float32),
        ),
        mesh=mesh,
        scratch_types=[
            pltpu.VMEM((D, CW + 1), jnp.float32),
            pltpu.VMEM((D, CW + 1), jnp.float32),
            pltpu.VMEM((OW, CHUNK), jnp.float32),
            pltpu.VMEM((OW, CHUNK), jnp.float32),
            pltpu.SemaphoreType.DMA,
            pltpu.SemaphoreType.DMA,
            pltpu.SemaphoreType.DMA,
            pltpu.SemaphoreType.DMA,
        ],
        compiler_params=pltpu.CompilerParams(needs_layout_passes=False),
    )
    def k(wt_in, wt_out, wr_in, wr_out, p_in, p_out,
          in0, in1, ou0, ou1, si0, si1, so0, so1):
        wid = lax.axis_index("s") * NC + lax.axis_index("c")
        ins, ous = (in0, in1), (ou0, ou1)
        sis, sos = (si0, si1), (so0, so1)
        row_base = [jax.lax.iota(jnp.int32, 16) + 16 * t for t in range(4)]

        def start_in(t, b):
            m = t * NW + wid

            @pl.when(m < NB)
            def _():
                pltpu.async_copy(wt_in.at[:, pl.ds(m * CW, CW)],
                                 ins[b].at[:, pl.ds(0, CW)], sis[b])

            @pl.when(jnp.logical_and(m >= NB, m < TOT))
            def _():
                pltpu.async_copy(wt_out.at[:, pl.ds((m - NB) * CW, CW)],
                                 ins[b].at[:, pl.ds(0, CW)], sis[b])

        def wait_in(b):
            pltpu.make_async_copy(wt_in.at[:, pl.ds(0, CW)],
                                  ins[b].at[:, pl.ds(0, CW)], sis[b]).wait()

        def start_out(t, b):
            m = t * NW + wid

            @pl.when(m < NB)
            def _():
                pltpu.async_copy(ous[b], p_in.at[pl.ds(m * OW, OW)], sos[b])

            @pl.when(jnp.logical_and(m >= NB, m < TOT))
            def _():
                pltpu.async_copy(
                    ous[b], p_out.at[pl.ds((m - NB) * OW, OW)], sos[b])

        def wait_out(b):
            pltpu.make_async_copy(ous[b], p_in.at[pl.ds(0, OW)], sos[b]).wait()

        def transpose(b, nrows):
            @plsc.parallel_loop(0, nrows, unroll=8)
            def _(u):
                c0 = jnp.full((16,), 2 * u, jnp.int32)
                for t in range(4):
                    ous[b][u, pl.ds(16 * t, 16)] = plsc.load_gather(
                        ins[b], [row_base[t], c0])
                    ous[b][u, pl.ds(64 + 16 * t, 16)] = plsc.load_gather(
                        ins[b], [row_base[t], c0 + 1])

        NREM = (V - VFULL) // 2

        def do_rem(wr, pout, owner):
            @pl.when(wid == owner)
            def _():
                pltpu.sync_copy(wr, ins[0].at[:, pl.ds(0, CHUNK)])
                transpose(0, NREM)
                pltpu.sync_copy(ous[0].at[pl.ds(0, NREM)],
                                pout.at[pl.ds(VPAIR - NREM, NREM)])
        do_rem(wr_in, p_in, 0)
        do_rem(wr_out, p_out, 1)

        start_in(0, 0)
        start_in(1, 1)

        def step(i, _):
            for b in range(2):
                t = 2 * i + b

                @pl.when(t >= 2)
                def _():
                    @pl.when((t - 2) * NW + wid < TOT)
                    def _():
                        wait_out(b)

                @pl.when(t * NW + wid < TOT)
                def _():
                    wait_in(b)
                plsc.subcore_barrier()

                @pl.when(t * NW + wid < TOT)
                def _():
                    transpose(b, OW)
                    start_out(t, b)
                    start_in(t + 2, b)
            return _
        lax.fori_loop(0, TMAX // 2, step, None)

    return k


def _sc_gather(ip2d, op2d, on2d, P_in, P_out):
    mesh = plsc.VectorSubcoreMesh(core_axis_name="c", subcore_axis_name="s")

    @functools.partial(
        pl.kernel,
        out_type=(
            jax.ShapeDtypeStruct((B, CHUNK), jnp.float32),
            jax.ShapeDtypeStruct((B, CHUNK), jnp.float32),
            jax.ShapeDtypeStruct((B * K, CHUNK), jnp.float32),
        ),
        mesh=mesh,
        scratch_types=[
            pltpu.VMEM((CA, CHUNK), jnp.int32),
            pltpu.VMEM((CP, CHUNK), jnp.int32),
            pltpu.VMEM((CN, CHUNK), jnp.int32),
            pltpu.VMEM((CHUNK, CHUNK), jnp.float32),
            pltpu.SemaphoreType.DMA,
        ],
    )
    def k(ip_hbm, op_hbm, on_hbm, pin_hbm, pout_hbm,
          out_in, out_pos, out_neg, ia_v, ip_v, in_v, rows_v, sem):
        wid = lax.axis_index("s") * NC + lax.axis_index("c")
        pltpu.sync_copy(ip_hbm.at[pl.ds(wid * CA, CA)], ia_v)
        pltpu.sync_copy(op_hbm.at[pl.ds(wid * CP, CP)], ip_v)
        pltpu.sync_copy(on_hbm.at[pl.ds(wid * CN, CN)], in_v)

        for c in range(CA):
            pltpu.async_copy(pin_hbm.at[ia_v.at[c]], rows_v, sem).wait()
            pltpu.sync_copy(rows_v, out_in.at[pl.ds((wid * CA + c) * CHUNK, CHUNK)])
        for c in range(CP):
            pltpu.async_copy(pout_hbm.at[ip_v.at[c]], rows_v, sem).wait()
            pltpu.sync_copy(rows_v, out_pos.at[pl.ds((wid * CP + c) * CHUNK, CHUNK)])

        def body(c, _):
            pltpu.async_copy(pout_hbm.at[in_v.at[c]], rows_v, sem).wait()
            pltpu.sync_copy(rows_v, out_neg.at[pl.ds((wid * CN + c) * CHUNK, CHUNK)])
            return _
        lax.fori_loop(0, CN, body, None)

    return k(ip2d, op2d, on2d, P_in, P_out)


_BB = 512


def _log_sigmoid(v):
    return jnp.minimum(v, 0.0) - jnp.log1p(jnp.exp(-jnp.abs(v)))


def _loss_body(in2_ref, pos2_ref, neg2_ref, ip_ref, op_ref, on_ref, out_ref):
    i = pl.program_id(0)
    ipb = ip_ref[...].reshape(_BB, 1)
    opb = op_ref[...].reshape(_BB, 1)
    onb = on_ref[...]
    x2 = in2_ref[...]
    p2 = pos2_ref[...]
    n2 = neg2_ref[...].reshape(_BB, K, CHUNK)
    x = jnp.where((ipb & 1) == 1, x2[:, D:], x2[:, :D])
    p = jnp.where((opb & 1) == 1, p2[:, D:], p2[:, :D])
    n = jnp.where((onb[..., None] & 1) == 1, n2[:, :, D:], n2[:, :, :D])
    pos_sim = jnp.clip(jnp.sum(x * p, axis=1), -10.0, 10.0)
    neg_sim = jnp.clip(jnp.sum(n * x[:, None, :], axis=2), -10.0, 10.0)
    tot = jnp.sum(_log_sigmoid(pos_sim)) + jnp.sum(_log_sigmoid(-neg_sim))

    @pl.when(i == 0)
    def _():
        out_ref[...] = jnp.zeros_like(out_ref)
    out_ref[...] += tot[None, None]


def _tc_loss(in2, pos2, neg2, input_pos, output_pos, output_neg):
    grid = (B // _BB,)
    res = pl.pallas_call(
        _loss_body,
        grid=grid,
        in_specs=[
            pl.BlockSpec((_BB, CHUNK), lambda i: (i, 0)),
            pl.BlockSpec((_BB, CHUNK), lambda i: (i, 0)),
            pl.BlockSpec((_BB * K, CHUNK), lambda i: (i, 0)),
            pl.BlockSpec((1, 1, _BB), lambda i: (i, 0, 0)),
            pl.BlockSpec((1, 1, _BB), lambda i: (i, 0, 0)),
            pl.BlockSpec((_BB, K), lambda i: (i, 0)),
        ],
        out_specs=pl.BlockSpec((1, 1), lambda i: (0, 0)),
        out_shape=jax.ShapeDtypeStruct((1, 1), jnp.float32),
        compiler_params=pltpu.CompilerParams(vmem_limit_bytes=100 * 1024 * 1024),
    )(in2, pos2, neg2, input_pos.reshape(-1, 1, _BB), output_pos.reshape(-1, 1, _BB),
      output_neg)
    return res


def kernel(input_pos, output_pos, output_neg, W_in, W_out):
    pad = ((0, 0), (0, CHUNK - (V - VFULL)))
    wr_in = jnp.pad(W_in.T[:, VFULL:], pad)
    wr_out = jnp.pad(W_out.T[:, VFULL:], pad)
    P_in, P_out = _sc_detile()(W_in.T, W_out.T, wr_in, wr_out)
    ip2d = (input_pos >> 1).reshape(-1, CHUNK)
    op2d = (output_pos >> 1).reshape(-1, CHUNK)
    on2d = (output_neg >> 1).reshape(-1, CHUNK)
    in2, pos2, neg2 = _sc_gather(ip2d, op2d, on2d, P_in, P_out)
    total = _tc_loss(in2, pos2, neg2, input_pos, output_pos, output_neg)
    return -total[0, 0] / B


# speedup vs baseline: 1.9817x; 1.9817x over previous
import functools

import jax
import jax.numpy as jnp
from jax import lax
from jax.experimental import pallas as pl
from jax.experimental.pallas import tpu as pltpu
from jax.experimental.pallas import tpu_sc as plsc

B = 16384
D = 64
K = 20
V = 1000000
NC = 2
NS = 16
NW = NC * NS
CHUNK = 128

VFULL = (V // CHUNK) * CHUNK
NBLK = V // CHUNK
VPAIR = V // 2

CA = B // NW // CHUNK
CP = B // NW // CHUNK
CN = B * K // NW // CHUNK


def _sc_detile():
    mesh = plsc.VectorSubcoreMesh(core_axis_name="c", subcore_axis_name="s")
    CW = 384
    OW = CW // 2
    NB = V // CW
    TOT = 2 * NB
    TMAX = 2 * ((TOT // NW) // 2 + 2)

    @functools.partial(
        pl.kernel,
        out_type=(
            jax.ShapeDtypeStruct((VPAIR, CHUNK), jnp.float32),
            jax.ShapeDtypeStruct((VPAIR, CHUNK), jnp.float32),
        ),
        mesh=mesh,
        scratch_types=[
            pltpu.VMEM((D, CW), jnp.float32),
            pltpu.VMEM((D, CW), jnp.float32),
            pltpu.VMEM((OW, CHUNK), jnp.float32),
            pltpu.VMEM((OW, CHUNK), jnp.float32),
            pltpu.SemaphoreType.DMA,
            pltpu.SemaphoreType.DMA,
            pltpu.SemaphoreType.DMA,
            pltpu.SemaphoreType.DMA,
        ],
        compiler_params=pltpu.CompilerParams(needs_layout_passes=False),
    )
    def k(wt_in, wt_out, wr_in, wr_out, p_in, p_out,
          in0, in1, ou0, ou1, si0, si1, so0, so1):
        wid = lax.axis_index("s") * NC + lax.axis_index("c")
        ins, ous = (in0, in1), (ou0, ou1)
        sis, sos = (si0, si1), (so0, so1)
        row_base = [jax.lax.iota(jnp.int32, 16) + 16 * t for t in range(4)]

        def start_in(t, b):
            m = t * NW + wid

            @pl.when(m < NB)
            def _():
                pltpu.async_copy(wt_in.at[:, pl.ds(m * CW, CW)], ins[b], sis[b])

            @pl.when(jnp.logical_and(m >= NB, m < TOT))
            def _():
                pltpu.async_copy(
                    wt_out.at[:, pl.ds((m - NB) * CW, CW)], ins[b], sis[b])

        def wait_in(b):
            pltpu.make_async_copy(wt_in.at[:, pl.ds(0, CW)], ins[b], sis[b]).wait()

        def start_out(t, b):
            m = t * NW + wid

            @pl.when(m < NB)
            def _():
                pltpu.async_copy(ous[b], p_in.at[pl.ds(m * OW, OW)], sos[b])

            @pl.when(jnp.logical_and(m >= NB, m < TOT))
            def _():
                pltpu.async_copy(
                    ous[b], p_out.at[pl.ds((m - NB) * OW, OW)], sos[b])

        def wait_out(b):
            pltpu.make_async_copy(ous[b], p_in.at[pl.ds(0, OW)], sos[b]).wait()

        def transpose(b, nrows):
            lanes = jax.lax.iota(jnp.int32, 16)
            perms = [(lanes + kk) % 16 for kk in range(16)]
            for g in range(4):
                rv = lanes + 16 * g

                @plsc.parallel_loop(0, 2 * nrows // 16, unroll=2)
                def _(ci):
                    c0v = jnp.full((16,), ci * 16, jnp.int32)
                    for kk in range(16):
                        cvec = perms[kk] + c0v
                        val = plsc.load_gather(ins[b], [rv, cvec])
                        uvec = cvec >> 1
                        pvec = (perms[kk] & 1) * 64 + rv
                        plsc.store_scatter(ous[b], [uvec, pvec], val)

        NREM = (V - VFULL) // 2

        def do_rem(wr, pout, owner):
            @pl.when(wid == owner)
            def _():
                pltpu.sync_copy(wr, ins[0].at[:, pl.ds(0, CHUNK)])
                transpose(0, NREM)
                pltpu.sync_copy(ous[0].at[pl.ds(0, NREM)],
                                pout.at[pl.ds(VPAIR - NREM, NREM)])
        do_rem(wr_in, p_in, 0)
        do_rem(wr_out, p_out, 1)

        start_in(0, 0)
        start_in(1, 1)

        def step(i, _):
            for b in range(2):
                t = 2 * i + b

                @pl.when(t >= 2)
                def _():
                    @pl.when((t - 2) * NW + wid < TOT)
                    def _():
                        wait_out(b)

                @pl.when(t * NW + wid < TOT)
                def _():
                    wait_in(b)
                    transpose(b, OW)
                    start_out(t, b)
                    start_in(t + 2, b)
            return _
        lax.fori_loop(0, TMAX // 2, step, None)

    return k


def _sc_gather(ip2d, op2d, on2d, P_in, P_out):
    mesh = plsc.VectorSubcoreMesh(core_axis_name="c", subcore_axis_name="s")

    @functools.partial(
        pl.kernel,
        out_type=(
            jax.ShapeDtypeStruct((B, CHUNK), jnp.float32),
            jax.ShapeDtypeStruct((B, CHUNK), jnp.float32),
            jax.ShapeDtypeStruct((B * K, CHUNK), jnp.float32),
        ),
        mesh=mesh,
        scratch_types=[
            pltpu.VMEM((CA, CHUNK), jnp.int32),
            pltpu.VMEM((CP, CHUNK), jnp.int32),
            pltpu.VMEM((CN, CHUNK), jnp.int32),
            pltpu.VMEM((CHUNK, CHUNK), jnp.float32),
            pltpu.SemaphoreType.DMA,
        ],
    )
    def k(ip_hbm, op_hbm, on_hbm, pin_hbm, pout_hbm,
          out_in, out_pos, out_neg, ia_v, ip_v, in_v, rows_v, sem):
        wid = lax.axis_index("s") * NC + lax.axis_index("c")
        pltpu.sync_copy(ip_hbm.at[pl.ds(wid * CA, CA)], ia_v)
        pltpu.sync_copy(op_hbm.at[pl.ds(wid * CP, CP)], ip_v)
        pltpu.sync_copy(on_hbm.at[pl.ds(wid * CN, CN)], in_v)

        for c in range(CA):
            pltpu.async_copy(pin_hbm.at[ia_v.at[c]], rows_v, sem).wait()
            pltpu.sync_copy(rows_v, out_in.at[pl.ds((wid * CA + c) * CHUNK, CHUNK)])
        for c in range(CP):
            pltpu.async_copy(pout_hbm.at[ip_v.at[c]], rows_v, sem).wait()
            pltpu.sync_copy(rows_v, out_pos.at[pl.ds((wid * CP + c) * CHUNK, CHUNK)])

        def body(c, _):
            pltpu.async_copy(pout_hbm.at[in_v.at[c]], rows_v, sem).wait()
            pltpu.sync_copy(rows_v, out_neg.at[pl.ds((wid * CN + c) * CHUNK, CHUNK)])
            return _
        lax.fori_loop(0, CN, body, None)

    return k(ip2d, op2d, on2d, P_in, P_out)


_BB = 512


def _log_sigmoid(v):
    return jnp.minimum(v, 0.0) - jnp.log1p(jnp.exp(-jnp.abs(v)))


def _loss_body(in2_ref, pos2_ref, neg2_ref, ip_ref, op_ref, on_ref, out_ref):
    i = pl.program_id(0)
    ipb = ip_ref[...].reshape(_BB, 1)
    opb = op_ref[...].reshape(_BB, 1)
    onb = on_ref[...]
    x2 = in2_ref[...]
    p2 = pos2_ref[...]
    n2 = neg2_ref[...].reshape(_BB, K, CHUNK)
    x = jnp.where((ipb & 1) == 1, x2[:, D:], x2[:, :D])
    p = jnp.where((opb & 1) == 1, p2[:, D:], p2[:, :D])
    n = jnp.where((onb[..., None] & 1) == 1, n2[:, :, D:], n2[:, :, :D])
    pos_sim = jnp.clip(jnp.sum(x * p, axis=1), -10.0, 10.0)
    neg_sim = jnp.clip(jnp.sum(n * x[:, None, :], axis=2), -10.0, 10.0)
    tot = jnp.sum(_log_sigmoid(pos_sim)) + jnp.sum(_log_sigmoid(-neg_sim))

    @pl.when(i == 0)
    def _():
        out_ref[...] = jnp.zeros_like(out_ref)
    out_ref[...] += tot[None, None]


def _tc_loss(in2, pos2, neg2, input_pos, output_pos, output_neg):
    grid = (B // _BB,)
    res = pl.pallas_call(
        _loss_body,
        grid=grid,
        in_specs=[
            pl.BlockSpec((_BB, CHUNK), lambda i: (i, 0)),
            pl.BlockSpec((_BB, CHUNK), lambda i: (i, 0)),
            pl.BlockSpec((_BB * K, CHUNK), lambda i: (i, 0)),
            pl.BlockSpec((1, 1, _BB), lambda i: (i, 0, 0)),
            pl.BlockSpec((1, 1, _BB), lambda i: (i, 0, 0)),
            pl.BlockSpec((_BB, K), lambda i: (i, 0)),
        ],
        out_specs=pl.BlockSpec((1, 1), lambda i: (0, 0)),
        out_shape=jax.ShapeDtypeStruct((1, 1), jnp.float32),
        compiler_params=pltpu.CompilerParams(vmem_limit_bytes=100 * 1024 * 1024),
    )(in2, pos2, neg2, input_pos.reshape(-1, 1, _BB), output_pos.reshape(-1, 1, _BB),
      output_neg)
    return res


def kernel(input_pos, output_pos, output_neg, W_in, W_out):
    pad = ((0, 0), (0, CHUNK - (V - VFULL)))
    wr_in = jnp.pad(W_in.T[:, VFULL:], pad)
    wr_out = jnp.pad(W_out.T[:, VFULL:], pad)
    P_in, P_out = _sc_detile()(W_in.T, W_out.T, wr_in, wr_out)
    ip2d = (input_pos >> 1).reshape(-1, CHUNK)
    op2d = (output_pos >> 1).reshape(-1, CHUNK)
    on2d = (output_neg >> 1).reshape(-1, CHUNK)
    in2, pos2, neg2 = _sc_gather(ip2d, op2d, on2d, P_in, P_out)
    total = _tc_loss(in2, pos2, neg2, input_pos, output_pos, output_neg)
    return -total[0, 0] / B


# speedup vs baseline: 2.1199x; 1.0697x over previous
import functools

import jax
import jax.numpy as jnp
from jax import lax
from jax.experimental import pallas as pl
from jax.experimental.pallas import tpu as pltpu
from jax.experimental.pallas import tpu_sc as plsc

B = 16384
D = 64
K = 20
V = 1000000
NC = 2
NS = 16
NW = NC * NS
CHUNK = 128

VFULL = (V // CHUNK) * CHUNK
NBLK = V // CHUNK
VPAIR = V // 2

CA = B // NW // CHUNK
CP = B // NW // CHUNK
CN = B * K // NW // CHUNK


def _sc_detile():
    mesh = plsc.VectorSubcoreMesh(core_axis_name="c", subcore_axis_name="s")
    CW = 384
    OW = CW // 2
    NB = V // CW
    TOT = 2 * NB
    TMAX = 2 * ((TOT // NW) // 2 + 2)

    @functools.partial(
        pl.kernel,
        out_type=(
            jax.ShapeDtypeStruct((VPAIR, CHUNK), jnp.float32),
            jax.ShapeDtypeStruct((VPAIR, CHUNK), jnp.float32),
        ),
        mesh=mesh,
        scratch_types=[
            pltpu.VMEM((D, CW), jnp.float32),
            pltpu.VMEM((D, CW), jnp.float32),
            pltpu.VMEM((OW, CHUNK), jnp.float32),
            pltpu.VMEM((OW, CHUNK), jnp.float32),
            pltpu.SemaphoreType.DMA,
            pltpu.SemaphoreType.DMA,
            pltpu.SemaphoreType.DMA,
            pltpu.SemaphoreType.DMA,
        ],
        compiler_params=pltpu.CompilerParams(needs_layout_passes=False),
    )
    def k(wt_in, wt_out, wr_in, wr_out, p_in, p_out,
          in0, in1, ou0, ou1, si0, si1, so0, so1):
        wid = lax.axis_index("s") * NC + lax.axis_index("c")
        ins, ous = (in0, in1), (ou0, ou1)
        sis, sos = (si0, si1), (so0, so1)
        row_base = [jax.lax.iota(jnp.int32, 16) + 16 * t for t in range(4)]

        def start_in(t, b):
            m = t * NW + wid

            @pl.when(m < NB)
            def _():
                pltpu.async_copy(wt_in.at[:, pl.ds(m * CW, CW)], ins[b], sis[b])

            @pl.when(jnp.logical_and(m >= NB, m < TOT))
            def _():
                pltpu.async_copy(
                    wt_out.at[:, pl.ds((m - NB) * CW, CW)], ins[b], sis[b])

        def wait_in(b):
            pltpu.make_async_copy(wt_in.at[:, pl.ds(0, CW)], ins[b], sis[b]).wait()

        def start_out(t, b):
            m = t * NW + wid

            @pl.when(m < NB)
            def _():
                pltpu.async_copy(ous[b], p_in.at[pl.ds(m * OW, OW)], sos[b])

            @pl.when(jnp.logical_and(m >= NB, m < TOT))
            def _():
                pltpu.async_copy(
                    ous[b], p_out.at[pl.ds((m - NB) * OW, OW)], sos[b])

        def wait_out(b):
            pltpu.make_async_copy(ous[b], p_in.at[pl.ds(0, OW)], sos[b]).wait()

        def transpose(b, nrows):
            lanes = jax.lax.iota(jnp.int32, 16)
            perms = [(lanes + kk) % 16 for kk in range(16)]
            for g in range(4):
                rv = lanes + 16 * g

                @plsc.parallel_loop(0, 2 * nrows // 16, unroll=2)
                def _(ci):
                    c0v = jnp.full((16,), ci * 16, jnp.int32)
                    for kk in range(16):
                        cvec = perms[kk] + c0v
                        val = plsc.load_gather(ins[b], [rv, cvec])
                        uvec = cvec >> 1
                        pvec = (perms[kk] & 1) * 64 + rv
                        plsc.store_scatter(ous[b], [uvec, pvec], val)

        NREM = (V - VFULL) // 2

        def do_rem(wr, pout, owner):
            @pl.when(wid == owner)
            def _():
                pltpu.sync_copy(wr, ins[0].at[:, pl.ds(0, CHUNK)])
                transpose(0, NREM)
                pltpu.sync_copy(ous[0].at[pl.ds(0, NREM)],
                                pout.at[pl.ds(VPAIR - NREM, NREM)])
        do_rem(wr_in, p_in, 0)
        do_rem(wr_out, p_out, 1)

        start_in(0, 0)
        start_in(1, 1)

        def step(i, _):
            for b in range(2):
                t = 2 * i + b

                @pl.when(t >= 2)
                def _():
                    @pl.when((t - 2) * NW + wid < TOT)
                    def _():
                        wait_out(b)

                @pl.when(t * NW + wid < TOT)
                def _():
                    wait_in(b)
                    transpose(b, OW)
                    start_out(t, b)
                    start_in(t + 2, b)
            return _
        lax.fori_loop(0, TMAX // 2, step, None)

    return k


def _sc_gather(ip2d, op2d, on2d, P_in, P_out):
    mesh = plsc.VectorSubcoreMesh(core_axis_name="c", subcore_axis_name="s")

    @functools.partial(
        pl.kernel,
        out_type=(
            jax.ShapeDtypeStruct((B, CHUNK), jnp.float32),
            jax.ShapeDtypeStruct((B, CHUNK), jnp.float32),
            jax.ShapeDtypeStruct((B * K, CHUNK), jnp.float32),
        ),
        mesh=mesh,
        scratch_types=[
            pltpu.VMEM((CA, CHUNK), jnp.int32),
            pltpu.VMEM((CP, CHUNK), jnp.int32),
            pltpu.VMEM((CN, CHUNK), jnp.int32),
            pltpu.VMEM((CHUNK, CHUNK), jnp.float32),
            pltpu.SemaphoreType.DMA,
        ],
    )
    def k(ip_hbm, op_hbm, on_hbm, pin_hbm, pout_hbm,
          out_in, out_pos, out_neg, ia_v, ip_v, in_v, rows_v, sem):
        wid = lax.axis_index("s") * NC + lax.axis_index("c")
        pltpu.sync_copy(ip_hbm.at[pl.ds(wid * CA, CA)], ia_v)
        pltpu.sync_copy(op_hbm.at[pl.ds(wid * CP, CP)], ip_v)
        pltpu.sync_copy(on_hbm.at[pl.ds(wid * CN, CN)], in_v)

        for c in range(CA):
            pltpu.async_copy(pin_hbm.at[ia_v.at[c]], rows_v, sem).wait()
            pltpu.sync_copy(rows_v, out_in.at[pl.ds((wid * CA + c) * CHUNK, CHUNK)])
        for c in range(CP):
            pltpu.async_copy(pout_hbm.at[ip_v.at[c]], rows_v, sem).wait()
            pltpu.sync_copy(rows_v, out_pos.at[pl.ds((wid * CP + c) * CHUNK, CHUNK)])

        def body(c, _):
            pltpu.async_copy(pout_hbm.at[in_v.at[c]], rows_v, sem).wait()
            pltpu.sync_copy(rows_v, out_neg.at[pl.ds((wid * CN + c) * CHUNK, CHUNK)])
            return _
        lax.fori_loop(0, CN, body, None)

    return k(ip2d, op2d, on2d, P_in, P_out)


_BB = 512


def _log_sigmoid(v):
    return jnp.minimum(v, 0.0) - jnp.log1p(jnp.exp(-jnp.abs(v)))


def _loss_body(in2_ref, pos2_ref, neg2_ref, ip_ref, op_ref, on_ref, out_ref):
    i = pl.program_id(0)
    ipb = ip_ref[...].reshape(_BB, 1)
    opb = op_ref[...].reshape(_BB, 1)
    onb = on_ref[...]
    x2 = in2_ref[...]
    p2 = pos2_ref[...]
    n2 = neg2_ref[...].reshape(_BB, K, CHUNK)
    lane_hi = jax.lax.broadcasted_iota(jnp.int32, (1, CHUNK), 1) >= D
    hx = (ipb & 1) == 1
    xF = jnp.where(lane_hi == hx, x2, pltpu.roll(x2, D, 1))
    hp = (opb & 1) == 1
    pF = jnp.where(lane_hi == hp, p2, pltpu.roll(p2, D, 1))
    pos_sim = 0.5 * jnp.sum(xF * pF, axis=1)
    prod = n2 * xF[:, None, :]
    hn = (onb[:, :, None] & 1) == 1
    neg_sim = jnp.sum(jnp.where(lane_hi[None] == hn, prod, 0.0), axis=2)
    pos_sim = jnp.clip(pos_sim, -10.0, 10.0)
    neg_sim = jnp.clip(neg_sim, -10.0, 10.0)
    tot = jnp.sum(_log_sigmoid(pos_sim)) + jnp.sum(_log_sigmoid(-neg_sim))

    @pl.when(i == 0)
    def _():
        out_ref[...] = jnp.zeros_like(out_ref)
    out_ref[...] += tot[None, None]


def _tc_loss(in2, pos2, neg2, input_pos, output_pos, output_neg):
    grid = (B // _BB,)
    res = pl.pallas_call(
        _loss_body,
        grid=grid,
        in_specs=[
            pl.BlockSpec((_BB, CHUNK), lambda i: (i, 0)),
            pl.BlockSpec((_BB, CHUNK), lambda i: (i, 0)),
            pl.BlockSpec((_BB * K, CHUNK), lambda i: (i, 0)),
            pl.BlockSpec((1, 1, _BB), lambda i: (i, 0, 0)),
            pl.BlockSpec((1, 1, _BB), lambda i: (i, 0, 0)),
            pl.BlockSpec((_BB, K), lambda i: (i, 0)),
        ],
        out_specs=pl.BlockSpec((1, 1), lambda i: (0, 0)),
        out_shape=jax.ShapeDtypeStruct((1, 1), jnp.float32),
        compiler_params=pltpu.CompilerParams(vmem_limit_bytes=100 * 1024 * 1024),
    )(in2, pos2, neg2, input_pos.reshape(-1, 1, _BB), output_pos.reshape(-1, 1, _BB),
      output_neg)
    return res


def kernel(input_pos, output_pos, output_neg, W_in, W_out):
    pad = ((0, 0), (0, CHUNK - (V - VFULL)))
    wr_in = jnp.pad(W_in.T[:, VFULL:], pad)
    wr_out = jnp.pad(W_out.T[:, VFULL:], pad)
    P_in, P_out = _sc_detile()(W_in.T, W_out.T, wr_in, wr_out)
    ip2d = (input_pos >> 1).reshape(-1, CHUNK)
    op2d = (output_pos >> 1).reshape(-1, CHUNK)
    on2d = (output_neg >> 1).reshape(-1, CHUNK)
    in2, pos2, neg2 = _sc_gather(ip2d, op2d, on2d, P_in, P_out)
    total = _tc_loss(in2, pos2, neg2, input_pos, output_pos, output_neg)
    return -total[0, 0] / B


# speedup vs baseline: 2.2671x; 1.0694x over previous
import functools

import jax
import jax.numpy as jnp
from jax import lax
from jax.experimental import pallas as pl
from jax.experimental.pallas import tpu as pltpu
from jax.experimental.pallas import tpu_sc as plsc

B = 16384
D = 64
K = 20
V = 1000000
NC = 2
NS = 16
NW = NC * NS
CHUNK = 128

VFULL = (V // CHUNK) * CHUNK
NBLK = V // CHUNK
VPAIR = V // 2

CA = B // NW // CHUNK
CP = B // NW // CHUNK
CN = B * K // NW // CHUNK


def _sc_detile():
    mesh = plsc.VectorSubcoreMesh(core_axis_name="c", subcore_axis_name="s")
    CW = 384
    OW = CW // 2
    NB = V // CW
    TOT = 2 * NB
    TMAX = 2 * ((TOT // NW) // 2 + 2)

    @functools.partial(
        pl.kernel,
        out_type=(
            jax.ShapeDtypeStruct((VPAIR, CHUNK), jnp.float32),
            jax.ShapeDtypeStruct((VPAIR, CHUNK), jnp.float32),
        ),
        mesh=mesh,
        scratch_types=[
            pltpu.VMEM((D, CW), jnp.float32),
            pltpu.VMEM((D, CW), jnp.float32),
            pltpu.VMEM((OW, CHUNK), jnp.float32),
            pltpu.VMEM((OW, CHUNK), jnp.float32),
            pltpu.SemaphoreType.DMA,
            pltpu.SemaphoreType.DMA,
            pltpu.SemaphoreType.DMA,
            pltpu.SemaphoreType.DMA,
        ],
        compiler_params=pltpu.CompilerParams(needs_layout_passes=False),
    )
    def k(wt_in, wt_out, wr_in, wr_out, p_in, p_out,
          in0, in1, ou0, ou1, si0, si1, so0, so1):
        wid = lax.axis_index("s") * NC + lax.axis_index("c")
        ins, ous = (in0, in1), (ou0, ou1)
        sis, sos = (si0, si1), (so0, so1)
        row_base = [jax.lax.iota(jnp.int32, 16) + 16 * t for t in range(4)]

        def start_in(t, b):
            m = t * NW + wid

            @pl.when(m < NB)
            def _():
                pltpu.async_copy(wt_in.at[:, pl.ds(m * CW, CW)], ins[b], sis[b])

            @pl.when(jnp.logical_and(m >= NB, m < TOT))
            def _():
                pltpu.async_copy(
                    wt_out.at[:, pl.ds((m - NB) * CW, CW)], ins[b], sis[b])

        def wait_in(b):
            pltpu.make_async_copy(wt_in.at[:, pl.ds(0, CW)], ins[b], sis[b]).wait()

        def start_out(t, b):
            m = t * NW + wid

            @pl.when(m < NB)
            def _():
                pltpu.async_copy(ous[b], p_in.at[pl.ds(m * OW, OW)], sos[b])

            @pl.when(jnp.logical_and(m >= NB, m < TOT))
            def _():
                pltpu.async_copy(
                    ous[b], p_out.at[pl.ds((m - NB) * OW, OW)], sos[b])

        def wait_out(b):
            pltpu.make_async_copy(ous[b], p_in.at[pl.ds(0, OW)], sos[b]).wait()

        def transpose(b, nrows):
            lanes = jax.lax.iota(jnp.int32, 16)
            perms = [(lanes + kk) % 16 for kk in range(16)]
            for g in range(4):
                rv = lanes + 16 * g

                @plsc.parallel_loop(0, 2 * nrows // 16, unroll=2)
                def _(ci):
                    c0v = jnp.full((16,), ci * 16, jnp.int32)
                    for kk in range(16):
                        cvec = perms[kk] + c0v
                        val = plsc.load_gather(ins[b], [rv, cvec])
                        uvec = cvec >> 1
                        pvec = (perms[kk] & 1) * 64 + rv
                        plsc.store_scatter(ous[b], [uvec, pvec], val)

        NREM = (V - VFULL) // 2

        def do_rem(wr, pout, owner):
            @pl.when(wid == owner)
            def _():
                pltpu.sync_copy(wr, ins[0].at[:, pl.ds(0, CHUNK)])
                transpose(0, NREM)
                pltpu.sync_copy(ous[0].at[pl.ds(0, NREM)],
                                pout.at[pl.ds(VPAIR - NREM, NREM)])
        do_rem(wr_in, p_in, 0)
        do_rem(wr_out, p_out, 1)

        start_in(0, 0)
        start_in(1, 1)

        def step(i, _):
            for b in range(2):
                t = 2 * i + b

                @pl.when(t >= 2)
                def _():
                    @pl.when((t - 2) * NW + wid < TOT)
                    def _():
                        wait_out(b)

                @pl.when(t * NW + wid < TOT)
                def _():
                    wait_in(b)
                    transpose(b, OW)
                    start_out(t, b)
                    start_in(t + 2, b)
            return _
        lax.fori_loop(0, TMAX // 2, step, None)

    return k


CT = CA + CP + CN


def _sc_gather(cidx, P_in, P_out):
    mesh = plsc.VectorSubcoreMesh(core_axis_name="c", subcore_axis_name="s")

    @functools.partial(
        pl.kernel,
        out_type=(
            jax.ShapeDtypeStruct((B, CHUNK), jnp.float32),
            jax.ShapeDtypeStruct((B, CHUNK), jnp.float32),
            jax.ShapeDtypeStruct((B * K, CHUNK), jnp.float32),
        ),
        mesh=mesh,
        scratch_types=[
            pltpu.VMEM((CT, CHUNK), jnp.int32),
            pltpu.VMEM((CHUNK, CHUNK), jnp.float32),
            pltpu.VMEM((CHUNK, CHUNK), jnp.float32),
            pltpu.VMEM((CHUNK, CHUNK), jnp.float32),
            pltpu.SemaphoreType.DMA,
            pltpu.SemaphoreType.DMA,
            pltpu.SemaphoreType.DMA,
            pltpu.SemaphoreType.DMA,
            pltpu.SemaphoreType.DMA,
            pltpu.SemaphoreType.DMA,
        ],
    )
    def k(cidx_hbm, pin_hbm, pout_hbm, out_in, out_pos, out_neg,
          idx_v, r0, r1, r2, g0, g1, g2, s0, s1, s2):
        wid = lax.axis_index("s") * NC + lax.axis_index("c")
        rows, gs, ss = (r0, r1, r2), (g0, g1, g2), (s0, s1, s2)
        pltpu.sync_copy(cidx_hbm.at[pl.ds(wid * CT, CT)], idx_v)

        def start_g(c, b):
            @pl.when(c < CA)
            def _():
                pltpu.async_copy(pin_hbm.at[idx_v.at[c]], rows[b], gs[b])

            @pl.when(c >= CA)
            def _():
                pltpu.async_copy(pout_hbm.at[idx_v.at[c]], rows[b], gs[b])

        def wait_g(b):
            pltpu.make_async_copy(pin_hbm.at[idx_v.at[0]], rows[b], gs[b]).wait()

        def start_s(c, b):
            @pl.when(c < CA)
            def _():
                pltpu.async_copy(
                    rows[b], out_in.at[pl.ds((wid * CA + c) * CHUNK, CHUNK)], ss[b])

            @pl.when(jnp.logical_and(c >= CA, c < CA + CP))
            def _():
                pltpu.async_copy(
                    rows[b],
                    out_pos.at[pl.ds((wid * CP + c - CA) * CHUNK, CHUNK)], ss[b])

            @pl.when(c >= CA + CP)
            def _():
                pltpu.async_copy(
                    rows[b],
                    out_neg.at[pl.ds((wid * CN + c - CA - CP) * CHUNK, CHUNK)],
                    ss[b])

        def wait_s(b):
            pltpu.make_async_copy(rows[b], out_in.at[pl.ds(0, CHUNK)], ss[b]).wait()

        def step(ii, _):
            for b in range(3):
                c = 3 * ii + b

                @pl.when(jnp.logical_and(c >= 3, c - 3 < CT))
                def _():
                    wait_s(b)

                @pl.when(c < CT)
                def _():
                    start_g(c, b)

                cm = c - 1
                bp = (b + 2) % 3

                @pl.when(jnp.logical_and(cm >= 0, cm < CT))
                def _():
                    wait_g(bp)
                    start_s(cm, bp)
            return _
        lax.fori_loop(0, (CT + 3) // 3 + 1, step, None)

    return k(cidx, P_in, P_out)


_BB = 512


def _log_sigmoid(v):
    return jnp.minimum(v, 0.0) - jnp.log1p(jnp.exp(-jnp.abs(v)))


def _loss_body(in2_ref, pos2_ref, neg2_ref, ip_ref, op_ref, on_ref, out_ref):
    i = pl.program_id(0)
    ipb = ip_ref[...].reshape(_BB, 1)
    opb = op_ref[...].reshape(_BB, 1)
    onb = on_ref[...]
    x2 = in2_ref[...]
    p2 = pos2_ref[...]
    n2 = neg2_ref[...].reshape(_BB, K, CHUNK)
    lane_hi = jax.lax.broadcasted_iota(jnp.int32, (1, CHUNK), 1) >= D
    hx = (ipb & 1) == 1
    xF = jnp.where(lane_hi == hx, x2, pltpu.roll(x2, D, 1))
    hp = (opb & 1) == 1
    pF = jnp.where(lane_hi == hp, p2, pltpu.roll(p2, D, 1))
    pos_sim = 0.5 * jnp.sum(xF * pF, axis=1)
    prod = n2 * xF[:, None, :]
    hn = (onb[:, :, None] & 1) == 1
    neg_sim = jnp.sum(jnp.where(lane_hi[None] == hn, prod, 0.0), axis=2)
    pos_sim = jnp.clip(pos_sim, -10.0, 10.0)
    neg_sim = jnp.clip(neg_sim, -10.0, 10.0)
    tot = jnp.sum(_log_sigmoid(pos_sim)) + jnp.sum(_log_sigmoid(-neg_sim))

    @pl.when(i == 0)
    def _():
        out_ref[...] = jnp.zeros_like(out_ref)
    out_ref[...] += tot[None, None]


def _tc_loss(in2, pos2, neg2, input_pos, output_pos, output_neg):
    grid = (B // _BB,)
    res = pl.pallas_call(
        _loss_body,
        grid=grid,
        in_specs=[
            pl.BlockSpec((_BB, CHUNK), lambda i: (i, 0)),
            pl.BlockSpec((_BB, CHUNK), lambda i: (i, 0)),
            pl.BlockSpec((_BB * K, CHUNK), lambda i: (i, 0)),
            pl.BlockSpec((1, 1, _BB), lambda i: (i, 0, 0)),
            pl.BlockSpec((1, 1, _BB), lambda i: (i, 0, 0)),
            pl.BlockSpec((_BB, K), lambda i: (i, 0)),
        ],
        out_specs=pl.BlockSpec((1, 1), lambda i: (0, 0)),
        out_shape=jax.ShapeDtypeStruct((1, 1), jnp.float32),
        compiler_params=pltpu.CompilerParams(vmem_limit_bytes=100 * 1024 * 1024),
    )(in2, pos2, neg2, input_pos.reshape(-1, 1, _BB), output_pos.reshape(-1, 1, _BB),
      output_neg)
    return res


def kernel(input_pos, output_pos, output_neg, W_in, W_out):
    pad = ((0, 0), (0, CHUNK - (V - VFULL)))
    wr_in = jnp.pad(W_in.T[:, VFULL:], pad)
    wr_out = jnp.pad(W_out.T[:, VFULL:], pad)
    P_in, P_out = _sc_detile()(W_in.T, W_out.T, wr_in, wr_out)
    ip3 = (input_pos >> 1).reshape(NW, CA, CHUNK)
    op3 = (output_pos >> 1).reshape(NW, CP, CHUNK)
    on3 = (output_neg >> 1).reshape(NW, CN, CHUNK)
    cidx = jnp.concatenate([ip3, op3, on3], axis=1).reshape(NW * CT, CHUNK)
    in2, pos2, neg2 = _sc_gather(cidx, P_in, P_out)
    total = _tc_loss(in2, pos2, neg2, input_pos, output_pos, output_neg)
    return -total[0, 0] / B


# speedup vs baseline: 2.7418x; 1.2094x over previous
import functools

import jax
import jax.numpy as jnp
from jax import lax
from jax.experimental import pallas as pl
from jax.experimental.pallas import tpu as pltpu
from jax.experimental.pallas import tpu_sc as plsc

B = 16384
D = 64
K = 20
V = 1000000
NC = 2
NS = 16
NW = NC * NS
CHUNK = 128

VFULL = (V // CHUNK) * CHUNK
NBLK = V // CHUNK
VPAIR = V // 2

CA = B // NW // CHUNK
CP = B // NW // CHUNK
CN = B * K // NW // CHUNK


def _sc_detile():
    mesh = plsc.VectorSubcoreMesh(core_axis_name="c", subcore_axis_name="s")
    CW = 384
    OW = CW // 2
    NB = V // CW
    TOT = 2 * NB
    TMAX = 2 * ((TOT // NW) // 2 + 2)

    @functools.partial(
        pl.kernel,
        out_type=(
            jax.ShapeDtypeStruct((VPAIR, CHUNK), jnp.float32),
            jax.ShapeDtypeStruct((VPAIR, CHUNK), jnp.float32),
        ),
        mesh=mesh,
        scratch_types=[
            pltpu.VMEM((D, CW), jnp.float32),
            pltpu.VMEM((D, CW), jnp.float32),
            pltpu.VMEM((OW, CHUNK), jnp.float32),
            pltpu.VMEM((OW, CHUNK), jnp.float32),
            pltpu.SemaphoreType.DMA,
            pltpu.SemaphoreType.DMA,
            pltpu.SemaphoreType.DMA,
            pltpu.SemaphoreType.DMA,
        ],
        compiler_params=pltpu.CompilerParams(needs_layout_passes=False),
    )
    def k(wt_in, wt_out, wr_in, wr_out, p_in, p_out,
          in0, in1, ou0, ou1, si0, si1, so0, so1):
        wid = lax.axis_index("s") * NC + lax.axis_index("c")
        ins, ous = (in0, in1), (ou0, ou1)
        sis, sos = (si0, si1), (so0, so1)
        row_base = [jax.lax.iota(jnp.int32, 16) + 16 * t for t in range(4)]

        def start_in(t, b):
            m = t * NW + wid

            @pl.when(m < NB)
            def _():
                pltpu.async_copy(wt_in.at[:, pl.ds(m * CW, CW)], ins[b], sis[b])

            @pl.when(jnp.logical_and(m >= NB, m < TOT))
            def _():
                pltpu.async_copy(
                    wt_out.at[:, pl.ds((m - NB) * CW, CW)], ins[b], sis[b])

        def wait_in(b):
            pltpu.make_async_copy(wt_in.at[:, pl.ds(0, CW)], ins[b], sis[b]).wait()

        def start_out(t, b):
            m = t * NW + wid

            @pl.when(m < NB)
            def _():
                pltpu.async_copy(ous[b], p_in.at[pl.ds(m * OW, OW)], sos[b])

            @pl.when(jnp.logical_and(m >= NB, m < TOT))
            def _():
                pltpu.async_copy(
                    ous[b], p_out.at[pl.ds((m - NB) * OW, OW)], sos[b])

        def wait_out(b):
            pltpu.make_async_copy(ous[b], p_in.at[pl.ds(0, OW)], sos[b]).wait()

        def transpose(b, nrows):
            lanes = jax.lax.iota(jnp.int32, 16)
            perms = [(lanes + kk) % 16 for kk in range(16)]
            for g in range(4):
                rv = lanes + 16 * g

                @plsc.parallel_loop(0, 2 * nrows // 16, unroll=4)
                def _(ci):
                    c0v = jnp.full((16,), ci * 16, jnp.int32)
                    for kk in range(16):
                        cvec = perms[kk] + c0v
                        val = plsc.load_gather(ins[b], [rv, cvec])
                        uvec = cvec >> 1
                        pvec = (perms[kk] & 1) * 64 + rv
                        plsc.store_scatter(ous[b], [uvec, pvec], val)

        NREM = (V - VFULL) // 2

        def do_rem(wr, pout, owner):
            @pl.when(wid == owner)
            def _():
                pltpu.sync_copy(wr, ins[0].at[:, pl.ds(0, CHUNK)])
                transpose(0, NREM)
                pltpu.sync_copy(ous[0].at[pl.ds(0, NREM)],
                                pout.at[pl.ds(VPAIR - NREM, NREM)])
        do_rem(wr_in, p_in, 0)
        do_rem(wr_out, p_out, 1)

        start_in(0, 0)
        start_in(1, 1)

        def step(i, _):
            for b in range(2):
                t = 2 * i + b

                @pl.when(t >= 2)
                def _():
                    @pl.when((t - 2) * NW + wid < TOT)
                    def _():
                        wait_out(b)

                @pl.when(t * NW + wid < TOT)
                def _():
                    wait_in(b)
                    transpose(b, OW)
                    start_out(t, b)
                    start_in(t + 2, b)
            return _
        lax.fori_loop(0, TMAX // 2, step, None)

    return k


CT = CA + CP + CN


def _sc_gather(cidx, P_in, P_out):
    mesh = plsc.VectorSubcoreMesh(core_axis_name="c", subcore_axis_name="s")

    @functools.partial(
        pl.kernel,
        out_type=(
            jax.ShapeDtypeStruct((B, CHUNK), jnp.float32),
            jax.ShapeDtypeStruct((B, CHUNK), jnp.float32),
            jax.ShapeDtypeStruct((B * K, CHUNK), jnp.float32),
        ),
        mesh=mesh,
        scratch_types=[
            pltpu.VMEM((CT, CHUNK), jnp.int32),
            pltpu.VMEM((CHUNK, CHUNK), jnp.float32),
            pltpu.VMEM((CHUNK, CHUNK), jnp.float32),
            pltpu.VMEM((CHUNK, CHUNK), jnp.float32),
            pltpu.SemaphoreType.DMA,
            pltpu.SemaphoreType.DMA,
            pltpu.SemaphoreType.DMA,
            pltpu.SemaphoreType.DMA,
            pltpu.SemaphoreType.DMA,
            pltpu.SemaphoreType.DMA,
        ],
    )
    def k(cidx_hbm, pin_hbm, pout_hbm, out_in, out_pos, out_neg,
          idx_v, r0, r1, r2, g0, g1, g2, s0, s1, s2):
        wid = lax.axis_index("s") * NC + lax.axis_index("c")
        rows, gs, ss = (r0, r1, r2), (g0, g1, g2), (s0, s1, s2)
        pltpu.sync_copy(cidx_hbm.at[pl.ds(wid * CT, CT)], idx_v)

        def start_g(c, b):
            @pl.when(c < CA)
            def _():
                pltpu.async_copy(pin_hbm.at[idx_v.at[c]], rows[b], gs[b])

            @pl.when(c >= CA)
            def _():
                pltpu.async_copy(pout_hbm.at[idx_v.at[c]], rows[b], gs[b])

        def wait_g(b):
            pltpu.make_async_copy(pin_hbm.at[idx_v.at[0]], rows[b], gs[b]).wait()

        def start_s(c, b):
            @pl.when(c < CA)
            def _():
                pltpu.async_copy(
                    rows[b], out_in.at[pl.ds((wid * CA + c) * CHUNK, CHUNK)], ss[b])

            @pl.when(jnp.logical_and(c >= CA, c < CA + CP))
            def _():
                pltpu.async_copy(
                    rows[b],
                    out_pos.at[pl.ds((wid * CP + c - CA) * CHUNK, CHUNK)], ss[b])

            @pl.when(c >= CA + CP)
            def _():
                pltpu.async_copy(
                    rows[b],
                    out_neg.at[pl.ds((wid * CN + c - CA - CP) * CHUNK, CHUNK)],
                    ss[b])

        def wait_s(b):
            pltpu.make_async_copy(rows[b], out_in.at[pl.ds(0, CHUNK)], ss[b]).wait()

        def step(ii, _):
            for b in range(3):
                c = 3 * ii + b

                @pl.when(jnp.logical_and(c >= 3, c - 3 < CT))
                def _():
                    wait_s(b)

                @pl.when(c < CT)
                def _():
                    start_g(c, b)

                cm = c - 1
                bp = (b + 2) % 3

                @pl.when(jnp.logical_and(cm >= 0, cm < CT))
                def _():
                    wait_g(bp)
                    start_s(cm, bp)
            return _
        lax.fori_loop(0, (CT + 3) // 3 + 1, step, None)

    return k(cidx, P_in, P_out)


_BB = 512


def _log_sigmoid(v):
    return jnp.minimum(v, 0.0) - jnp.log1p(jnp.exp(-jnp.abs(v)))


def _loss_body(in2_ref, pos2_ref, neg2_ref, ip_ref, op_ref, on_ref, out_ref):
    i = pl.program_id(0)
    ipb = ip_ref[...].reshape(_BB, 1)
    opb = op_ref[...].reshape(_BB, 1)
    onb = on_ref[...]
    x2 = in2_ref[...]
    p2 = pos2_ref[...]
    n2 = neg2_ref[...].reshape(_BB, K, CHUNK)
    lane_hi = jax.lax.broadcasted_iota(jnp.int32, (1, CHUNK), 1) >= D
    hx = (ipb & 1) == 1
    xF = jnp.where(lane_hi == hx, x2, pltpu.roll(x2, D, 1))
    hp = (opb & 1) == 1
    pF = jnp.where(lane_hi == hp, p2, pltpu.roll(p2, D, 1))
    pos_sim = 0.5 * jnp.sum(xF * pF, axis=1)
    prod = n2 * xF[:, None, :]
    hn = (onb[:, :, None] & 1) == 1
    neg_sim = jnp.sum(jnp.where(lane_hi[None] == hn, prod, 0.0), axis=2)
    pos_sim = jnp.clip(pos_sim, -10.0, 10.0)
    neg_sim = jnp.clip(neg_sim.T, -10.0, 10.0)
    tot = jnp.sum(_log_sigmoid(pos_sim)) + jnp.sum(_log_sigmoid(-neg_sim))

    @pl.when(i == 0)
    def _():
        out_ref[...] = jnp.zeros_like(out_ref)
    out_ref[...] += tot[None, None]


def _tc_loss(in2, pos2, neg2, input_pos, output_pos, output_neg):
    grid = (B // _BB,)
    res = pl.pallas_call(
        _loss_body,
        grid=grid,
        in_specs=[
            pl.BlockSpec((_BB, CHUNK), lambda i: (i, 0)),
            pl.BlockSpec((_BB, CHUNK), lambda i: (i, 0)),
            pl.BlockSpec((_BB * K, CHUNK), lambda i: (i, 0)),
            pl.BlockSpec((1, 1, _BB), lambda i: (i, 0, 0)),
            pl.BlockSpec((1, 1, _BB), lambda i: (i, 0, 0)),
            pl.BlockSpec((_BB, K), lambda i: (i, 0)),
        ],
        out_specs=pl.BlockSpec((1, 1), lambda i: (0, 0)),
        out_shape=jax.ShapeDtypeStruct((1, 1), jnp.float32),
        compiler_params=pltpu.CompilerParams(vmem_limit_bytes=100 * 1024 * 1024),
    )(in2, pos2, neg2, input_pos.reshape(-1, 1, _BB), output_pos.reshape(-1, 1, _BB),
      output_neg)
    return res


def kernel(input_pos, output_pos, output_neg, W_in, W_out):
    pad = ((0, 0), (0, CHUNK - (V - VFULL)))
    wr_in = jnp.pad(W_in.T[:, VFULL:], pad)
    wr_out = jnp.pad(W_out.T[:, VFULL:], pad)
    P_in, P_out = _sc_detile()(W_in.T, W_out.T, wr_in, wr_out)
    ip3 = (input_pos >> 1).reshape(NW, CA, CHUNK)
    op3 = (output_pos >> 1).reshape(NW, CP, CHUNK)
    on3 = (output_neg >> 1).reshape(NW, CN, CHUNK)
    cidx = jnp.concatenate([ip3, op3, on3], axis=1).reshape(NW * CT, CHUNK)
    in2, pos2, neg2 = _sc_gather(cidx, P_in, P_out)
    total = _tc_loss(in2, pos2, neg2, input_pos, output_pos, output_neg)
    return -total[0, 0] / B


# speedup vs baseline: 2.9417x; 1.0729x over previous
import functools

import jax
import jax.numpy as jnp
from jax import lax
from jax.experimental import pallas as pl
from jax.experimental.pallas import tpu as pltpu
from jax.experimental.pallas import tpu_sc as plsc

B = 16384
D = 64
K = 20
V = 1000000
NC = 2
NS = 16
NW = NC * NS
CHUNK = 128

VFULL = (V // CHUNK) * CHUNK
NBLK = V // CHUNK
VPAIR = V // 2

CA = B // NW // CHUNK
CP = B // NW // CHUNK
CN = B * K // NW // CHUNK


def _sc_detile():
    mesh = plsc.VectorSubcoreMesh(core_axis_name="c", subcore_axis_name="s")
    CW = 384
    OW = CW // 2
    NB = V // CW
    TOT = 2 * NB
    TMAX = 2 * ((TOT // NW) // 2 + 2)

    @functools.partial(
        pl.kernel,
        out_type=(
            jax.ShapeDtypeStruct((VPAIR, CHUNK), jnp.float32),
            jax.ShapeDtypeStruct((VPAIR, CHUNK), jnp.float32),
        ),
        mesh=mesh,
        scratch_types=[
            pltpu.VMEM((D, CW), jnp.float32),
            pltpu.VMEM((D, CW), jnp.float32),
            pltpu.VMEM((OW, CHUNK), jnp.float32),
            pltpu.VMEM((OW, CHUNK), jnp.float32),
            pltpu.SemaphoreType.DMA,
            pltpu.SemaphoreType.DMA,
            pltpu.SemaphoreType.DMA,
            pltpu.SemaphoreType.DMA,
        ],
        compiler_params=pltpu.CompilerParams(needs_layout_passes=False),
    )
    def k(wt_in, wt_out, wr_in, wr_out, p_in, p_out,
          in0, in1, ou0, ou1, si0, si1, so0, so1):
        wid = lax.axis_index("s") * NC + lax.axis_index("c")
        ins, ous = (in0, in1), (ou0, ou1)
        sis, sos = (si0, si1), (so0, so1)
        row_base = [jax.lax.iota(jnp.int32, 16) + 16 * t for t in range(4)]

        def start_in(t, b):
            m = t * NW + wid

            @pl.when(m < NB)
            def _():
                pltpu.async_copy(wt_in.at[:, pl.ds(m * CW, CW)], ins[b], sis[b])

            @pl.when(jnp.logical_and(m >= NB, m < TOT))
            def _():
                pltpu.async_copy(
                    wt_out.at[:, pl.ds((m - NB) * CW, CW)], ins[b], sis[b])

        def wait_in(b):
            pltpu.make_async_copy(wt_in.at[:, pl.ds(0, CW)], ins[b], sis[b]).wait()

        def start_out(t, b):
            m = t * NW + wid

            @pl.when(m < NB)
            def _():
                pltpu.async_copy(ous[b], p_in.at[pl.ds(m * OW, OW)], sos[b])

            @pl.when(jnp.logical_and(m >= NB, m < TOT))
            def _():
                pltpu.async_copy(
                    ous[b], p_out.at[pl.ds((m - NB) * OW, OW)], sos[b])

        def wait_out(b):
            pltpu.make_async_copy(ous[b], p_in.at[pl.ds(0, OW)], sos[b]).wait()

        def transpose(b, nrows):
            lanes = jax.lax.iota(jnp.int32, 16)
            perms = [(lanes + kk) % 16 for kk in range(16)]
            for g in range(4):
                rv = lanes + 16 * g

                @plsc.parallel_loop(0, 2 * nrows // 16, unroll=4)
                def _(ci):
                    c0v = jnp.full((16,), ci * 16, jnp.int32)
                    for kk in range(16):
                        cvec = perms[kk] + c0v
                        val = plsc.load_gather(ins[b], [rv, cvec])
                        uvec = cvec >> 1
                        pvec = (perms[kk] & 1) * 64 + rv
                        plsc.store_scatter(ous[b], [uvec, pvec], val)

        NREM = (V - VFULL) // 2

        def do_rem(wr, pout, owner):
            @pl.when(wid == owner)
            def _():
                pltpu.sync_copy(wr, ins[0].at[:, pl.ds(0, CHUNK)])
                transpose(0, NREM)
                pltpu.sync_copy(ous[0].at[pl.ds(0, NREM)],
                                pout.at[pl.ds(VPAIR - NREM, NREM)])
        do_rem(wr_in, p_in, 0)
        do_rem(wr_out, p_out, 1)

        start_in(0, 0)
        start_in(1, 1)

        def step(i, _):
            for b in range(2):
                t = 2 * i + b

                @pl.when(t >= 2)
                def _():
                    @pl.when((t - 2) * NW + wid < TOT)
                    def _():
                        wait_out(b)

                @pl.when(t * NW + wid < TOT)
                def _():
                    wait_in(b)
                    transpose(b, OW)
                    start_out(t, b)
                    start_in(t + 2, b)
            return _
        lax.fori_loop(0, TMAX // 2, step, None)

    return k


CT = CA + CP + CN


def _sc_gather(cidx, P_in, P_out, ca, cp, cn, bs, ctp):
    mesh = plsc.VectorSubcoreMesh(core_axis_name="c", subcore_axis_name="s")

    @functools.partial(
        pl.kernel,
        out_type=(
            jax.ShapeDtypeStruct((bs, CHUNK), jnp.float32),
            jax.ShapeDtypeStruct((bs, CHUNK), jnp.float32),
            jax.ShapeDtypeStruct((bs * K, CHUNK), jnp.float32),
        ),
        mesh=mesh,
        scratch_types=[
            pltpu.VMEM((ctp, CHUNK), jnp.int32),
            pltpu.VMEM((CHUNK, CHUNK), jnp.float32),
            pltpu.VMEM((CHUNK, CHUNK), jnp.float32),
            pltpu.VMEM((CHUNK, CHUNK), jnp.float32),
            pltpu.SemaphoreType.DMA,
            pltpu.SemaphoreType.DMA,
            pltpu.SemaphoreType.DMA,
            pltpu.SemaphoreType.DMA,
            pltpu.SemaphoreType.DMA,
            pltpu.SemaphoreType.DMA,
        ],
    )
    def k(cidx_hbm, pin_hbm, pout_hbm, out_in, out_pos, out_neg,
          idx_v, r0, r1, r2, g0, g1, g2, s0, s1, s2):
        wid = lax.axis_index("s") * NC + lax.axis_index("c")
        rows, gs, ss = (r0, r1, r2), (g0, g1, g2), (s0, s1, s2)
        pltpu.sync_copy(cidx_hbm.at[pl.ds(wid * ctp, ctp)], idx_v)

        def start_g(c, b):
            @pl.when(c < ca)
            def _():
                pltpu.async_copy(pin_hbm.at[idx_v.at[c]], rows[b], gs[b])

            @pl.when(c >= ca)
            def _():
                pltpu.async_copy(pout_hbm.at[idx_v.at[c]], rows[b], gs[b])

        def wait_g(b):
            pltpu.make_async_copy(pin_hbm.at[idx_v.at[0]], rows[b], gs[b]).wait()

        def start_s(c, b):
            @pl.when(c < ca)
            def _():
                pltpu.async_copy(
                    rows[b], out_in.at[pl.ds((wid * ca + c) * CHUNK, CHUNK)], ss[b])

            @pl.when(jnp.logical_and(c >= ca, c < ca + cp))
            def _():
                pltpu.async_copy(
                    rows[b],
                    out_pos.at[pl.ds((wid * cp + c - ca) * CHUNK, CHUNK)], ss[b])

            @pl.when(c >= ca + cp)
            def _():
                pltpu.async_copy(
                    rows[b],
                    out_neg.at[pl.ds((wid * cn + c - ca - cp) * CHUNK, CHUNK)],
                    ss[b])

        def wait_s(b):
            pltpu.make_async_copy(rows[b], out_in.at[pl.ds(0, CHUNK)], ss[b]).wait()

        CTT = ca + cp + cn

        def step(ii, _):
            for b in range(3):
                c = 3 * ii + b

                @pl.when(jnp.logical_and(c >= 3, c - 3 < CTT))
                def _():
                    wait_s(b)

                @pl.when(c < CTT)
                def _():
                    start_g(c, b)

                cm = c - 1
                bp = (b + 2) % 3

                @pl.when(jnp.logical_and(cm >= 0, cm < CTT))
                def _():
                    wait_g(bp)
                    start_s(cm, bp)
            return _
        lax.fori_loop(0, (CTT + 3) // 3 + 1, step, None)

    return k(cidx, P_in, P_out)


_BB = 512


def _log_sigmoid(v):
    return jnp.minimum(v, 0.0) - jnp.log1p(jnp.exp(-jnp.abs(v)))


def _loss_body(in2_ref, pos2_ref, neg2_ref, ip_ref, op_ref, on_ref, out_ref):
    i = pl.program_id(0)
    ipb = ip_ref[...].reshape(_BB, 1)
    opb = op_ref[...].reshape(_BB, 1)
    onb = on_ref[...]
    x2 = in2_ref[...]
    p2 = pos2_ref[...]
    n2 = neg2_ref[...].reshape(_BB, K, CHUNK)
    lane_hi = jax.lax.broadcasted_iota(jnp.int32, (1, CHUNK), 1) >= D
    hx = (ipb & 1) == 1
    xF = jnp.where(lane_hi == hx, x2, pltpu.roll(x2, D, 1))
    hp = (opb & 1) == 1
    pF = jnp.where(lane_hi == hp, p2, pltpu.roll(p2, D, 1))
    pos_sim = 0.5 * jnp.sum(xF * pF, axis=1)
    prod = n2 * xF[:, None, :]
    hn = (onb[:, :, None] & 1) == 1
    neg_sim = jnp.sum(jnp.where(lane_hi[None] == hn, prod, 0.0), axis=2)
    pos_sim = jnp.clip(pos_sim, -10.0, 10.0)
    neg_sim = jnp.clip(neg_sim.T, -10.0, 10.0)
    tot = jnp.sum(_log_sigmoid(pos_sim)) + jnp.sum(_log_sigmoid(-neg_sim))

    @pl.when(i == 0)
    def _():
        out_ref[...] = jnp.zeros_like(out_ref)
    out_ref[...] += tot[None, None]


def _tc_loss(in2, pos2, neg2, input_pos, output_pos, output_neg, bs):
    grid = (bs // _BB,)
    res = pl.pallas_call(
        _loss_body,
        grid=grid,
        in_specs=[
            pl.BlockSpec((_BB, CHUNK), lambda i: (i, 0)),
            pl.BlockSpec((_BB, CHUNK), lambda i: (i, 0)),
            pl.BlockSpec((_BB * K, CHUNK), lambda i: (i, 0)),
            pl.BlockSpec((1, 1, _BB), lambda i: (i, 0, 0)),
            pl.BlockSpec((1, 1, _BB), lambda i: (i, 0, 0)),
            pl.BlockSpec((_BB, K), lambda i: (i, 0)),
        ],
        out_specs=pl.BlockSpec((1, 1), lambda i: (0, 0)),
        out_shape=jax.ShapeDtypeStruct((1, 1), jnp.float32),
        compiler_params=pltpu.CompilerParams(vmem_limit_bytes=100 * 1024 * 1024),
    )(in2, pos2, neg2, input_pos.reshape(-1, 1, _BB), output_pos.reshape(-1, 1, _BB),
      output_neg)
    return res


def kernel(input_pos, output_pos, output_neg, W_in, W_out):
    pad = ((0, 0), (0, CHUNK - (V - VFULL)))
    wr_in = jnp.pad(W_in.T[:, VFULL:], pad)
    wr_out = jnp.pad(W_out.T[:, VFULL:], pad)
    P_in, P_out = _sc_detile()(W_in.T, W_out.T, wr_in, wr_out)
    S = 4
    BS = B // S
    ca, cp, cn = BS // NW // CHUNK, BS // NW // CHUNK, BS * K // NW // CHUNK
    total = jnp.float32(0.0)
    for si in range(S):
        sl = slice(si * BS, (si + 1) * BS)
        ip3 = (input_pos[sl] >> 1).reshape(NW, ca, CHUNK)
        op3 = (output_pos[sl] >> 1).reshape(NW, cp, CHUNK)
        on3 = (output_neg[sl] >> 1).reshape(NW, cn, CHUNK)
        ct = ca + cp + cn
        ctp = -(-ct // 8) * 8
        padr = jnp.zeros((NW, ctp - ct, CHUNK), jnp.int32)
        cidx = jnp.concatenate([ip3, op3, on3, padr], axis=1).reshape(-1, CHUNK)
        in2, pos2, neg2 = _sc_gather(cidx, P_in, P_out, ca, cp, cn, BS, ctp)
        res = _tc_loss(in2, pos2, neg2, input_pos[sl], output_pos[sl],
                       output_neg[sl], BS)
        total = total + res[0, 0]
    return -total / B
